# baseline (speedup 1.0000x reference)
import functools

import jax
import jax.numpy as jnp
from jax import lax
from jax.experimental import pallas as pl
from jax.experimental.pallas import tpu as pltpu
from jax.experimental.pallas import tpu_sc as plsc

NC, NS, LANES = 2, 16, 16
NW = NC * NS

B, L, D = 4, 8192, 768
N = B * L
TOK_PER_W = N // NW
CHUNK = 64
NCHUNK = TOK_PER_W // CHUNK
NJ = D // LANES
LN_EPS = 1e-12


def _allsum16(v):
    iota = lax.iota(jnp.int32, LANES)
    for shift in (8, 4, 2, 1):
        idx = (iota + shift) & (LANES - 1)
        v = v + v.at[idx].get(mode="promise_in_bounds")
    return v


def _rsqrt16(v):
    i = lax.bitcast_convert_type(v, jnp.int32)
    i = jnp.int32(0x5F3759DF) - lax.shift_right_logical(i, 1)
    y = lax.bitcast_convert_type(i, jnp.float32)
    for _ in range(3):
        y = y * (1.5 - 0.5 * v * y * y)
    return y


def _sc_embed(ids, pos, tt, word_table, pos_table, type_table, ln_w, ln_b):
    mesh = plsc.VectorSubcoreMesh(core_axis_name="c", subcore_axis_name="s")

    @functools.partial(
        pl.kernel,
        mesh=mesh,
        out_type=jax.ShapeDtypeStruct((N, D), jnp.float32),
        scratch_types=[
            pltpu.VMEM((CHUNK,), jnp.int32),
            pltpu.VMEM((CHUNK,), jnp.int32),
            pltpu.VMEM((CHUNK,), jnp.float32),
            pltpu.VMEM((CHUNK,), jnp.int32),
            pltpu.VMEM((CHUNK, D), jnp.float32),
            pltpu.VMEM((CHUNK, D), jnp.float32),
            pltpu.VMEM((2, D), jnp.float32),
            pltpu.VMEM((D,), jnp.float32),
            pltpu.VMEM((D,), jnp.float32),
            pltpu.VMEM((D,), jnp.float32),
            pltpu.SemaphoreType.DMA,
            pltpu.SemaphoreType.DMA,
        ],
    )
    def k(ids_hbm, pos_hbm, tt_hbm, word_hbm, post_hbm, type_hbm, lnw_hbm,
          lnb_hbm, out_hbm, idw_v, idp_v, ttf_v, tti_v, rw_v, rp_v, ty_v,
          td_v, lnw_v, lnb_v, sem_w, sem_p):
        wid = lax.axis_index("s") * NC + lax.axis_index("c")

        pltpu.sync_copy(type_hbm, ty_v)
        pltpu.sync_copy(lnw_hbm, lnw_v)
        pltpu.sync_copy(lnb_hbm, lnb_v)
        for j in range(NJ):
            sl = pl.ds(j * LANES, LANES)
            td_v[sl] = ty_v[1, sl] - ty_v[0, sl]

        def chunk_body(c, carry):
            base = wid * TOK_PER_W + c * CHUNK
            pltpu.sync_copy(ids_hbm.at[pl.ds(base, CHUNK)], idw_v)
            pltpu.sync_copy(pos_hbm.at[pl.ds(base, CHUNK)], idp_v)
            pltpu.sync_copy(tt_hbm.at[pl.ds(base, CHUNK)], tti_v)
            cp_w = pltpu.async_copy(word_hbm.at[idw_v], rw_v, sem_w)
            cp_p = pltpu.async_copy(post_hbm.at[idp_v], rp_v, sem_p)
            for g in range(CHUNK // LANES):
                gsl = pl.ds(g * LANES, LANES)
                ttf_v[gsl] = tti_v[gsl].astype(jnp.float32)
            cp_w.wait()
            cp_p.wait()

            def tok_body(t, tc):
                g = t // LANES
                lane = t - g * LANES
                lane_v = lax.broadcast_in_dim(lane, (LANES,), ())
                ttb = ttf_v[pl.ds(g * LANES, LANES)].at[lane_v].get(
                    mode="promise_in_bounds")
                acc = jnp.zeros((LANES,), jnp.float32)
                acc2 = jnp.zeros((LANES,), jnp.float32)
                for j in range(NJ):
                    sl = pl.ds(j * LANES, LANES)
                    x = rw_v[t, sl] + rp_v[t, sl] + (ty_v[0, sl]
                                                     + ttb * td_v[sl])
                    rw_v[t, sl] = x
                    acc = acc + x
                    acc2 = acc2 + x * x
                mv = _allsum16(acc) * (1.0 / D)
                ex2 = _allsum16(acc2) * (1.0 / D)
                var = ex2 - mv * mv
                inv = _rsqrt16(var + LN_EPS)
                for j in range(NJ):
                    sl = pl.ds(j * LANES, LANES)
                    xn = (rw_v[t, sl] - mv) * inv
                    rw_v[t, sl] = xn * lnw_v[sl] + lnb_v[sl]
                return tc

            lax.fori_loop(0, CHUNK, tok_body, 0)
            pltpu.sync_copy(rw_v, out_hbm.at[pl.ds(base, CHUNK)])
            return carry

        lax.fori_loop(0, NCHUNK, chunk_body, 0)

    return k(ids, pos, tt, word_table, pos_table, type_table, ln_w, ln_b)


def kernel(input_ids, token_type_ids, position_ids, word_table, pos_table,
           type_table, ln_w, ln_b):
    ids = jnp.asarray(input_ids, jnp.int32).reshape(N)
    pos = jnp.asarray(position_ids, jnp.int32).reshape(N)
    tt = jnp.asarray(token_type_ids, jnp.int32).reshape(N)
    out = _sc_embed(ids, pos, tt,
                    word_table.astype(jnp.float32),
                    pos_table.astype(jnp.float32),
                    type_table.astype(jnp.float32),
                    ln_w.astype(jnp.float32), ln_b.astype(jnp.float32))
    return out.reshape(B, L, D)
